# Initial kernel scaffold; baseline (speedup 1.0000x reference)
import jax
import jax.numpy as jnp
from jax.experimental import pallas as pl


def kernel(atom_inputs, element_table, degree_table, ring_table, charge_table, aromatic_table, hybrid_table, hydrogen_table, func_tables, h_don_table, h_acc_table, ringsize_table, aroma_num_table, fused_if_table, het27_table, func_reduce_W, func_reduce_b, bond_env_W, bond_env_b, disc_W, disc_b):
    raise NotImplementedError("write your pallas kernel here")


# speedup vs baseline: 68.7175x; 68.7175x over previous
import numpy as np

import jax
import jax.numpy as jnp
from jax.experimental import pallas as pl
from jax.experimental.pallas import tpu as pltpu

_I32_MIN = -2147483648
_I32_MAX = 2147483647

_GROUPS = (
    (0,  0,  (6, 7, 8, 14, 15, 16), _I32_MIN, _I32_MAX),
    (1,  6,  tuple(range(7)),  0, 6),
    (5,  13, (-1, 0),          -1, 0),
    (2,  15, tuple(range(8)),  0, 7),
    (4,  23, (0, 1),           0, 1),
    (3,  25, tuple(range(6)),  0, 5),
    (6,  31, tuple(range(5)),  0, 4),
) + tuple(
    (7 + i, 36 + 2 * i, (0, 1), 0, 1) for i in range(18)
) + (
    (25, 72, (0, 1),           0, 1),
    (26, 74, (0, 1),           0, 1),
    (27, 76, (0, 3, 4, 5, 6, 7), _I32_MIN, _I32_MAX),
    (28, 82, tuple(range(5)),  0, 4),
    (29, 87, tuple(range(8)),  0, 7),
    (30, 95, tuple(range(27)), 0, 26),
)
_BIAS_LANE = 122


def _static_tables():
    lb = np.zeros((1, 32), np.int32)
    ub = np.zeros((1, 32), np.int32)
    s = np.zeros((32, 128), np.float32)
    r = np.full((1, 128), -9999.0, np.float32)
    for col, off, exp, lo, hi in _GROUPS:
        lb[0, col] = lo
        ub[0, col] = hi
        for j, v in enumerate(exp):
            s[col, off + j] = 1.0
            r[0, off + j] = float(v)
    r[0, _BIAS_LANE] = 0.0
    return jnp.asarray(lb), jnp.asarray(ub), jnp.asarray(s), jnp.asarray(r)


_LB, _UB, _S, _R = _static_tables()


def _build_weights(element_table, degree_table, ring_table, charge_table,
                   aromatic_table, hybrid_table, hydrogen_table, func_tables,
                   h_don_table, h_acc_table, ringsize_table, aroma_num_table,
                   fused_if_table, het27_table, func_reduce_W, func_reduce_b,
                   bond_env_W, bond_env_b, disc_W, disc_b):
    flag_rows = jnp.einsum("fab,fbc->fac", func_tables,
                           func_reduce_W.reshape(18, 2, 4)).reshape(36, 4)

    def pad(tbl, c0):
        return jnp.pad(tbl, ((0, 0), (c0, 52 - c0 - tbl.shape[1])))

    g_rows = [
        pad(element_table[1:7] - element_table[0:1], 0),
        pad(degree_table, 4),
        pad(ring_table, 8),
        pad(charge_table, 12),
        pad(aromatic_table, 16),
        pad(hybrid_table, 20),
        pad(hydrogen_table, 24),
        pad(flag_rows, 28),
        pad(h_don_table, 32),
        pad(h_acc_table, 34),
        pad(ringsize_table[0:6] - ringsize_table[6:7], 36),
        pad(aroma_num_table, 40),
        pad(fused_if_table, 44),
        pad(het27_table, 48),
        pad(element_table[0:1], 0) + pad(ringsize_table[6:7], 36)
        + pad(func_reduce_b[None, :], 28),
        jnp.zeros((6, 52), jnp.float32),
    ]
    g = jnp.concatenate(g_rows, axis=0)[:128]
    m_disc = g @ disc_W
    m_disc = m_disc.at[_BIAS_LANE].add(disc_b)
    m = jnp.pad(m_disc, ((0, 0), (0, 16)))
    m = m.at[_BIAS_LANE, 48:64].set(bond_env_b)
    wb = jnp.zeros((79, 64), jnp.float32).at[31:79, 48:64].set(bond_env_W)
    return m, wb


def _body(x_ref, lb_ref, ub_ref, s_ref, r_ref, m_ref, wb_ref, out_ref):
    x = x_ref[...]
    xi = x[:, 0:32].astype(jnp.int32)
    c = jnp.clip(xi, lb_ref[...], ub_ref[...]).astype(jnp.float32)
    tb = jnp.dot(c, s_ref[...], preferred_element_type=jnp.float32)
    onehot = (tb == r_ref[...]).astype(jnp.float32)
    acc = jnp.dot(onehot, m_ref[...], preferred_element_type=jnp.float32)
    acc = acc + jnp.dot(x, wb_ref[...], preferred_element_type=jnp.float32)
    out_ref[...] = acc


def _pick_block(n):
    for b in (4000, 2000, 1000, 500, 200, 100, 50, 25, 8):
        if n % b == 0 and b % 8 == 0:
            return b
    return None


def kernel(atom_inputs, element_table, degree_table, ring_table, charge_table,
           aromatic_table, hybrid_table, hydrogen_table, func_tables,
           h_don_table, h_acc_table, ringsize_table, aroma_num_table,
           fused_if_table, het27_table, func_reduce_W, func_reduce_b,
           bond_env_W, bond_env_b, disc_W, disc_b):
    m, wb = _build_weights(element_table, degree_table, ring_table,
                           charge_table, aromatic_table, hybrid_table,
                           hydrogen_table, func_tables, h_don_table,
                           h_acc_table, ringsize_table, aroma_num_table,
                           fused_if_table, het27_table, func_reduce_W,
                           func_reduce_b, bond_env_W, bond_env_b,
                           disc_W, disc_b)
    n = atom_inputs.shape[0]
    b = _pick_block(n)
    x = atom_inputs
    n_pad = n
    if b is None:
        b = 4000
        n_pad = ((n + b - 1) // b) * b
        x = jnp.pad(x, ((0, n_pad - n), (0, 0)))
    grid = n_pad // b

    full = lambda shape: pl.BlockSpec(shape, lambda i: (0, 0))
    out = pl.pallas_call(
        _body,
        grid=(grid,),
        in_specs=[
            pl.BlockSpec((b, 79), lambda i: (i, 0)),
            full((1, 32)), full((1, 32)), full((32, 128)), full((1, 128)),
            full((128, 64)), full((79, 64)),
        ],
        out_specs=pl.BlockSpec((b, 64), lambda i: (i, 0)),
        out_shape=jax.ShapeDtypeStruct((n_pad, 64), jnp.float32),
        compiler_params=pltpu.CompilerParams(
            dimension_semantics=("arbitrary",),
        ),
    )(x, _LB, _UB, _S, _R, m, wb)
    return out[:n]


# speedup vs baseline: 75.8226x; 1.1034x over previous
import numpy as np

import jax
import jax.numpy as jnp
from jax.experimental import pallas as pl
from jax.experimental.pallas import tpu as pltpu

_I32_MIN = -2147483648
_I32_MAX = 2147483647

_GROUPS = (
    (0,  0,  (6, 7, 8, 14, 15, 16), _I32_MIN, _I32_MAX),
    (1,  6,  tuple(range(7)),  0, 6),
    (5,  13, (-1, 0),          -1, 0),
    (2,  15, tuple(range(8)),  0, 7),
    (4,  23, (0, 1),           0, 1),
    (3,  25, tuple(range(6)),  0, 5),
    (6,  31, tuple(range(5)),  0, 4),
) + tuple(
    (7 + i, 36 + 2 * i, (0, 1), 0, 1) for i in range(18)
) + (
    (25, 72, (0, 1),           0, 1),
    (26, 74, (0, 1),           0, 1),
    (27, 76, (0, 3, 4, 5, 6, 7), _I32_MIN, _I32_MAX),
    (28, 82, tuple(range(5)),  0, 4),
    (29, 87, tuple(range(8)),  0, 7),
    (30, 95, tuple(range(27)), 0, 26),
)
_BIAS_LANE = 122


def _static_tables():
    lb = np.zeros((1, 32), np.int32)
    ub = np.zeros((1, 32), np.int32)
    s = np.zeros((32, 128), np.float32)
    r = np.full((1, 128), -9999.0, np.float32)
    for col, off, exp, lo, hi in _GROUPS:
        lb[0, col] = lo
        ub[0, col] = hi
        for j, v in enumerate(exp):
            s[col, off + j] = 1.0
            r[0, off + j] = float(v)
    r[0, _BIAS_LANE] = 0.0
    return jnp.asarray(lb), jnp.asarray(ub), jnp.asarray(s), jnp.asarray(r)


_LB, _UB, _S, _R = _static_tables()


def _build_weights(element_table, degree_table, ring_table, charge_table,
                   aromatic_table, hybrid_table, hydrogen_table, func_tables,
                   h_don_table, h_acc_table, ringsize_table, aroma_num_table,
                   fused_if_table, het27_table, func_reduce_W, func_reduce_b,
                   bond_env_W, bond_env_b, disc_W, disc_b):
    flag_rows = jnp.einsum("fab,fbc->fac", func_tables,
                           func_reduce_W.reshape(18, 2, 4)).reshape(36, 4)

    def pad(tbl, c0):
        return jnp.pad(tbl, ((0, 0), (c0, 52 - c0 - tbl.shape[1])))

    g_rows = [
        pad(element_table[1:7] - element_table[0:1], 0),
        pad(degree_table, 4),
        pad(ring_table, 8),
        pad(charge_table, 12),
        pad(aromatic_table, 16),
        pad(hybrid_table, 20),
        pad(hydrogen_table, 24),
        pad(flag_rows, 28),
        pad(h_don_table, 32),
        pad(h_acc_table, 34),
        pad(ringsize_table[0:6] - ringsize_table[6:7], 36),
        pad(aroma_num_table, 40),
        pad(fused_if_table, 44),
        pad(het27_table, 48),
        pad(element_table[0:1], 0) + pad(ringsize_table[6:7], 36)
        + pad(func_reduce_b[None, :], 28),
        jnp.zeros((6, 52), jnp.float32),
    ]
    g = jnp.concatenate(g_rows, axis=0)[:128]
    m_disc = g @ disc_W
    m_disc = m_disc.at[_BIAS_LANE].add(disc_b)
    m = jnp.pad(m_disc, ((0, 0), (0, 16)))
    m = m.at[_BIAS_LANE, 48:64].set(bond_env_b)
    wb = jnp.zeros((79, 64), jnp.float32).at[31:79, 48:64].set(bond_env_W)
    return m, wb


def _body(x_ref, lb_ref, ub_ref, s_ref, r_ref, m_ref, wb_ref, out_ref):
    x = x_ref[...]
    xi = x[:, 0:32].astype(jnp.int32)
    c = jnp.clip(xi, lb_ref[...], ub_ref[...]).astype(jnp.float32)
    tb = jnp.dot(c, s_ref[...], preferred_element_type=jnp.float32)
    onehot = (tb == r_ref[...]).astype(jnp.float32)
    acc = jnp.dot(onehot, m_ref[...], preferred_element_type=jnp.float32)
    acc = acc + jnp.dot(x, wb_ref[...], preferred_element_type=jnp.float32)
    out_ref[...] = acc


def _pick_block(n):
    for b in (20000, 4000, 2000, 1000, 500, 200, 100, 50, 25, 8):
        if n % b == 0 and b % 8 == 0:
            return b
    return None


def kernel(atom_inputs, element_table, degree_table, ring_table, charge_table,
           aromatic_table, hybrid_table, hydrogen_table, func_tables,
           h_don_table, h_acc_table, ringsize_table, aroma_num_table,
           fused_if_table, het27_table, func_reduce_W, func_reduce_b,
           bond_env_W, bond_env_b, disc_W, disc_b):
    m, wb = _build_weights(element_table, degree_table, ring_table,
                           charge_table, aromatic_table, hybrid_table,
                           hydrogen_table, func_tables, h_don_table,
                           h_acc_table, ringsize_table, aroma_num_table,
                           fused_if_table, het27_table, func_reduce_W,
                           func_reduce_b, bond_env_W, bond_env_b,
                           disc_W, disc_b)
    n = atom_inputs.shape[0]
    b = _pick_block(n)
    x = atom_inputs
    n_pad = n
    if b is None:
        b = 4000
        n_pad = ((n + b - 1) // b) * b
        x = jnp.pad(x, ((0, n_pad - n), (0, 0)))
    grid = n_pad // b

    full = lambda shape: pl.BlockSpec(shape, lambda i: (0, 0))
    out = pl.pallas_call(
        _body,
        grid=(grid,),
        in_specs=[
            pl.BlockSpec((b, 79), lambda i: (i, 0)),
            full((1, 32)), full((1, 32)), full((32, 128)), full((1, 128)),
            full((128, 64)), full((79, 64)),
        ],
        out_specs=pl.BlockSpec((b, 64), lambda i: (i, 0)),
        out_shape=jax.ShapeDtypeStruct((n_pad, 64), jnp.float32),
        compiler_params=pltpu.CompilerParams(
            dimension_semantics=("arbitrary",),
        ),
    )(x, _LB, _UB, _S, _R, m, wb)
    return out[:n]
